# Initial kernel scaffold; baseline (speedup 1.0000x reference)
import jax
import jax.numpy as jnp
from jax.experimental import pallas as pl


def kernel(x, edge_index, W1, b1, gamma, beta, Wmu, bmu, Wlv, blv):
    raise NotImplementedError("write your pallas kernel here")


# speedup vs baseline: 7.7424x; 7.7424x over previous
import functools

import jax
import jax.numpy as jnp
from jax import lax
from jax.experimental import pallas as pl
from jax.experimental.pallas import tpu as pltpu
from jax.experimental.pallas import tpu_sc as plsc

N = 10000
E = 160000
D = 256
NC = 2
NS = 16
CHUNK = 128
ACC_ROWS = 10240
E_PAD = 163840
STRIPE = ACC_ROWS // NS
LAST_STRIPE = N - (NS - 1) * STRIPE
ROWS_PT = E_PAD // NS // CHUNK
ROWS_PT_DEG = E_PAD // (NC * NS) // CHUNK
EPS = 1e-5
RB = 400
GRID = N // RB

_mesh = plsc.VectorSubcoreMesh(core_axis_name="c", subcore_axis_name="s")


def _deg_body(dst2d, out, idx_v, ones_v, zero_v, acc):
    c = lax.axis_index("c")
    s = lax.axis_index("s")
    z16 = jnp.zeros((16,), jnp.float32)
    o16 = jnp.ones((16,), jnp.float32)

    def fill(r, _):
        zero_v[r] = z16
        ones_v[r] = o16
        return 0

    lax.fori_loop(0, CHUNK, fill, 0)
    for j in range(STRIPE // CHUNK):
        pltpu.sync_copy(zero_v, acc.at[pl.ds(s * STRIPE + j * CHUNK, CHUNK)])
    wid = c * NS + s
    pltpu.sync_copy(dst2d.at[pl.ds(wid * ROWS_PT_DEG, ROWS_PT_DEG)], idx_v)
    plsc.subcore_barrier()

    def body(j, _):
        pltpu.sync_copy(ones_v, acc.at[idx_v.at[j]], add=True)
        return 0

    lax.fori_loop(0, ROWS_PT_DEG, body, 0)
    plsc.subcore_barrier()

    @pl.when(s < NS - 1)
    def _w():
        pltpu.sync_copy(acc.at[pl.ds(s * STRIPE, STRIPE)],
                        out.at[c, pl.ds(s * STRIPE, STRIPE)])

    @pl.when(s == NS - 1)
    def _wl():
        pltpu.sync_copy(acc.at[pl.ds((NS - 1) * STRIPE, LAST_STRIPE)],
                        out.at[c, pl.ds((NS - 1) * STRIPE, LAST_STRIPE)])


_deg_kernel = functools.partial(
    pl.kernel,
    out_type=jax.ShapeDtypeStruct((NC, N, 16), jnp.float32),
    mesh=_mesh,
    scratch_types=[
        pltpu.VMEM((ROWS_PT_DEG, CHUNK), jnp.int32),
        pltpu.VMEM((CHUNK, 16), jnp.float32),
        pltpu.VMEM((CHUNK, 16), jnp.float32),
        pltpu.VMEM_SHARED((ACC_ROWS, 16), jnp.float32),
    ],
)(_deg_body)


def _scat_body(ycat, src2d, dst2d, out, src_v, dst_v, gbuf, acc, sem):
    c = lax.axis_index("c")
    s = lax.axis_index("s")
    z16 = jnp.zeros((16,), jnp.float32)

    def zrow(r, _):
        for k in range(CHUNK // 16):
            gbuf[r, pl.ds(k * 16, 16)] = z16
        return 0

    lax.fori_loop(0, CHUNK, zrow, 0)
    for j in range(STRIPE // CHUNK):
        pltpu.sync_copy(gbuf, acc.at[pl.ds(s * STRIPE + j * CHUNK, CHUNK)])
    pltpu.sync_copy(src2d.at[pl.ds(s * ROWS_PT, ROWS_PT)], src_v)
    pltpu.sync_copy(dst2d.at[pl.ds(s * ROWS_PT, ROWS_PT)], dst_v)

    offv = jnp.broadcast_to(c * N, (16,)).astype(jnp.int32)

    def addoff(r, _):
        for k in range(CHUNK // 16):
            src_v[r, pl.ds(k * 16, 16)] += offv
        return 0

    lax.fori_loop(0, ROWS_PT, addoff, 0)
    plsc.subcore_barrier()

    def body(j, _):
        pltpu.async_copy(ycat.at[src_v.at[j]], gbuf, sem).wait()
        pltpu.sync_copy(gbuf, acc.at[dst_v.at[j]], add=True)
        return 0

    lax.fori_loop(0, ROWS_PT, body, 0)
    plsc.subcore_barrier()

    @pl.when(s < NS - 1)
    def _w():
        pltpu.sync_copy(acc.at[pl.ds(s * STRIPE, STRIPE)],
                        out.at[c, pl.ds(s * STRIPE, STRIPE)])

    @pl.when(s == NS - 1)
    def _wl():
        pltpu.sync_copy(acc.at[pl.ds((NS - 1) * STRIPE, LAST_STRIPE)],
                        out.at[c, pl.ds((NS - 1) * STRIPE, LAST_STRIPE)])


_scat_kernel = functools.partial(
    pl.kernel,
    out_type=jax.ShapeDtypeStruct((NC, N, 128), jnp.float32),
    mesh=_mesh,
    scratch_types=[
        pltpu.VMEM((ROWS_PT, CHUNK), jnp.int32),
        pltpu.VMEM((ROWS_PT, CHUNK), jnp.int32),
        pltpu.VMEM((CHUNK, 128), jnp.float32),
        pltpu.VMEM_SHARED((ACC_ROWS, 128), jnp.float32),
        pltpu.SemaphoreType.DMA,
    ],
)(_scat_body)


def _dinv_block(draw_ref, i):
    d = draw_ref[0, pl.ds(i * RB, RB), :] + draw_ref[1, pl.ds(i * RB, RB), :]
    return lax.rsqrt(d[:, 0:1] + 1.0)


def _stack_halves(y):
    return jnp.stack([y[:, :128], y[:, 128:]], axis=0)


def _tc1_body(x_ref, w_ref, draw_ref, y_ref):
    i = pl.program_id(0)
    dinv = _dinv_block(draw_ref, i)
    y = jnp.dot(x_ref[...], w_ref[...], preferred_element_type=jnp.float32) * dinv
    y_ref[...] = _stack_halves(y)


_tc1 = pl.pallas_call(
    _tc1_body,
    grid=(GRID,),
    in_specs=[
        pl.BlockSpec((RB, D), lambda i: (i, 0)),
        pl.BlockSpec((D, D), lambda i: (0, 0)),
        pl.BlockSpec((NC, N, 16), lambda i: (0, 0, 0)),
    ],
    out_specs=pl.BlockSpec((2, RB, 128), lambda i: (0, i, 0)),
    out_shape=jax.ShapeDtypeStruct((2, N, 128), jnp.float32),
)


def _conv_out_block(acc_ref, y_ref, b_ref, dinv):
    o = jnp.concatenate([acc_ref[0] + y_ref[0], acc_ref[1] + y_ref[1]], axis=1)
    return o * dinv + b_ref[...]


def _tc2a_body(acc_ref, y_ref, draw_ref, b_ref, st_ref):
    i = pl.program_id(0)
    o = _conv_out_block(acc_ref, y_ref, b_ref, _dinv_block(draw_ref, i))
    st = jnp.concatenate([jnp.sum(o, axis=0, keepdims=True),
                          jnp.sum(o * o, axis=0, keepdims=True),
                          jnp.zeros((6, D), jnp.float32)], axis=0)

    @pl.when(i == 0)
    def _z():
        st_ref[...] = jnp.zeros_like(st_ref)

    st_ref[...] += st


_tc2a = pl.pallas_call(
    _tc2a_body,
    grid=(GRID,),
    in_specs=[
        pl.BlockSpec((NC, RB, 128), lambda i: (0, i, 0)),
        pl.BlockSpec((2, RB, 128), lambda i: (0, i, 0)),
        pl.BlockSpec((NC, N, 16), lambda i: (0, 0, 0)),
        pl.BlockSpec((1, D), lambda i: (0, 0)),
    ],
    out_specs=pl.BlockSpec((8, D), lambda i: (0, 0)),
    out_shape=jax.ShapeDtypeStruct((8, D), jnp.float32),
)


def _tc2b_body(acc_ref, y_ref, draw_ref, b_ref, st_ref, g_ref, be_ref, y2_ref):
    i = pl.program_id(0)
    dinv = _dinv_block(draw_ref, i)
    o = _conv_out_block(acc_ref, y_ref, b_ref, dinv)
    mean = st_ref[0:1, :] * (1.0 / N)
    var = st_ref[1:2, :] * (1.0 / N) - mean * mean
    h = g_ref[...] * ((o - mean) * lax.rsqrt(var + EPS)) + be_ref[...]
    h = jnp.maximum(h, 0.0)
    y2_ref[...] = _stack_halves(h * dinv)


_tc2b = pl.pallas_call(
    _tc2b_body,
    grid=(GRID,),
    in_specs=[
        pl.BlockSpec((NC, RB, 128), lambda i: (0, i, 0)),
        pl.BlockSpec((2, RB, 128), lambda i: (0, i, 0)),
        pl.BlockSpec((NC, N, 16), lambda i: (0, 0, 0)),
        pl.BlockSpec((1, D), lambda i: (0, 0)),
        pl.BlockSpec((8, D), lambda i: (0, 0)),
        pl.BlockSpec((1, D), lambda i: (0, 0)),
        pl.BlockSpec((1, D), lambda i: (0, 0)),
    ],
    out_specs=pl.BlockSpec((2, RB, 128), lambda i: (0, i, 0)),
    out_shape=jax.ShapeDtypeStruct((2, N, 128), jnp.float32),
)


def _tc3_body(acc_ref, y2_ref, draw_ref, wmu_ref, bmu_ref, wlv_ref, blv_ref,
              mu_ref, lv_ref):
    i = pl.program_id(0)
    dinv = _dinv_block(draw_ref, i)
    sfull = jnp.concatenate([acc_ref[0] + y2_ref[0], acc_ref[1] + y2_ref[1]],
                            axis=1) * dinv
    mu_ref[...] = jnp.dot(sfull, wmu_ref[...],
                          preferred_element_type=jnp.float32) + bmu_ref[...]
    lv_ref[...] = jnp.dot(sfull, wlv_ref[...],
                          preferred_element_type=jnp.float32) + blv_ref[...]


_tc3 = pl.pallas_call(
    _tc3_body,
    grid=(GRID,),
    in_specs=[
        pl.BlockSpec((NC, RB, 128), lambda i: (0, i, 0)),
        pl.BlockSpec((2, RB, 128), lambda i: (0, i, 0)),
        pl.BlockSpec((NC, N, 16), lambda i: (0, 0, 0)),
        pl.BlockSpec((D, 128), lambda i: (0, 0)),
        pl.BlockSpec((1, 128), lambda i: (0, 0)),
        pl.BlockSpec((D, 128), lambda i: (0, 0)),
        pl.BlockSpec((1, 128), lambda i: (0, 0)),
    ],
    out_specs=[pl.BlockSpec((RB, 128), lambda i: (i, 0)),
               pl.BlockSpec((RB, 128), lambda i: (i, 0))],
    out_shape=(jax.ShapeDtypeStruct((N, 128), jnp.float32),
               jax.ShapeDtypeStruct((N, 128), jnp.float32)),
)


def kernel(x, edge_index, W1, b1, gamma, beta, Wmu, bmu, Wlv, blv):
    src = edge_index[0].astype(jnp.int32)
    dst = edge_index[1].astype(jnp.int32)
    srcp = jnp.concatenate([src, jnp.zeros((E_PAD - E,), jnp.int32)])
    dstp = jnp.concatenate([dst, jnp.full((E_PAD - E,), N, jnp.int32)])
    src2d = srcp.reshape(E_PAD // CHUNK, CHUNK)
    dst2d = dstp.reshape(E_PAD // CHUNK, CHUNK)

    b1r = b1.reshape(1, D)
    gr = gamma.reshape(1, D)
    ber = beta.reshape(1, D)

    draw = _deg_kernel(dst2d)
    ycat = _tc1(x, W1, draw)
    acc = _scat_kernel(ycat.reshape(2 * N, 128), src2d, dst2d)
    stats = _tc2a(acc, ycat, draw, b1r)
    y2cat = _tc2b(acc, ycat, draw, b1r, stats, gr, ber)
    acc2 = _scat_kernel(y2cat.reshape(2 * N, 128), src2d, dst2d)
    mu, lv = _tc3(acc2, y2cat, draw, Wmu, bmu.reshape(1, 128),
                  Wlv, blv.reshape(1, 128))
    return (mu, lv)
